# Initial kernel scaffold; baseline (speedup 1.0000x reference)
import functools

import jax
import jax.numpy as jnp
from jax import lax
from jax.experimental import pallas as pl
from jax.experimental.pallas import tpu as pltpu
from jax.experimental.pallas import tpu_sc as plsc

N = 10000
E = 160000
D = 256
HALF = 128
NSUB = 16
NCORE = 2
CH = 80
EPT = E // NSUB
NCHUNK = EPT // CH
RPT = N // NSUB
WB = 125


def _sc_mesh():
    return plsc.VectorSubcoreMesh(core_axis_name="c", subcore_axis_name="s")


def _sc_deg(dst):
    @functools.partial(
        pl.kernel,
        out_type=jax.ShapeDtypeStruct((N, 16), jnp.float32),
        mesh=_sc_mesh(),
        scratch_types=[
            pltpu.VMEM((CH,), jnp.int32),
            pltpu.VMEM((CH, 16), jnp.float32),
            pltpu.VMEM((RPT, 16), jnp.float32),
            pltpu.VMEM_SHARED((N, 16), jnp.float32),
        ],
    )
    def k(dst_hbm, out_hbm, didx_v, ones_v, buf_v, acc_sh):
        c = lax.axis_index("c")
        s = lax.axis_index("s")
        t0 = s * EPT

        def fill(i, _):
            ones_v[i, :] = jnp.ones((16,), jnp.float32)
            return 0

        lax.fori_loop(0, CH, fill, 0)

        def zero(i, _):
            buf_v[i, :] = jnp.zeros((16,), jnp.float32)
            return 0

        lax.fori_loop(0, RPT, zero, 0)
        pltpu.sync_copy(buf_v, acc_sh.at[pl.ds(s * RPT, RPT)])
        plsc.subcore_barrier()

        def chunk(j, _):
            pltpu.sync_copy(dst_hbm.at[pl.ds(t0 + j * CH, CH)], didx_v)
            pltpu.sync_copy(ones_v, acc_sh.at[didx_v], add=True)
            return 0

        lax.fori_loop(0, NCHUNK, chunk, 0)
        plsc.subcore_barrier()

        @pl.when(c == 0)
        def _():
            pltpu.sync_copy(acc_sh.at[pl.ds(s * RPT, RPT)], buf_v)
            pltpu.sync_copy(buf_v, out_hbm.at[pl.ds(s * RPT, RPT)])

    return k(dst)


def _sc_agg(h_flat, src, dst):
    @functools.partial(
        pl.kernel,
        out_type=jax.ShapeDtypeStruct((2 * N, HALF), jnp.float32),
        mesh=_sc_mesh(),
        scratch_types=[
            pltpu.VMEM((CH,), jnp.int32),
            pltpu.VMEM((CH,), jnp.int32),
            pltpu.VMEM((CH, HALF), jnp.float32),
            pltpu.VMEM((WB, HALF), jnp.float32),
            pltpu.VMEM_SHARED((N, HALF), jnp.float32),
            pltpu.SemaphoreType.DMA,
        ],
    )
    def k(h_hbm, src_hbm, dst_hbm, out_hbm, gidx_v, didx_v, rows_v, buf_v,
          acc_sh, sem):
        c = lax.axis_index("c")
        s = lax.axis_index("s")
        t0 = s * EPT
        off = c * N

        def zero(i, _):
            for jj in range(HALF // 16):
                buf_v[i, pl.ds(16 * jj, 16)] = jnp.zeros((16,), jnp.float32)
            return 0

        lax.fori_loop(0, WB, zero, 0)
        for t in range(RPT // WB):
            pltpu.sync_copy(buf_v, acc_sh.at[pl.ds(s * RPT + t * WB, WB)])
        plsc.subcore_barrier()

        def chunk(j, _):
            e0 = t0 + j * CH
            pltpu.sync_copy(src_hbm.at[pl.ds(e0, CH)], gidx_v)
            pltpu.sync_copy(dst_hbm.at[pl.ds(e0, CH)], didx_v)

            def xform(kk, _):
                sl = pl.ds(16 * kk, 16)
                gidx_v[sl] = gidx_v[sl] + jnp.broadcast_to(off, (16,))
                return 0

            lax.fori_loop(0, CH // 16, xform, 0)
            pltpu.async_copy(h_hbm.at[gidx_v], rows_v, sem).wait()
            pltpu.sync_copy(rows_v, acc_sh.at[didx_v], add=True)
            return 0

        lax.fori_loop(0, NCHUNK, chunk, 0)
        plsc.subcore_barrier()

        for t in range(RPT // WB):
            r0 = s * RPT + t * WB
            pltpu.sync_copy(acc_sh.at[pl.ds(r0, WB)], buf_v)
            pltpu.sync_copy(buf_v, out_hbm.at[pl.ds(off + r0, WB)])

    return k(h_flat, src, dst)


BM = 1000


def _tc_layer_body(a_ref, h_ref, deg_ref, wl_ref, bl_ref, wr_ref, o_ref):
    r = 1.0 / jnp.maximum(deg_ref[:, 0:1], 1.0)
    acc = jnp.dot(a_ref[0] * r, wl_ref[0:HALF, :],
                  preferred_element_type=jnp.float32)
    acc += jnp.dot(a_ref[1] * r, wl_ref[HALF:D, :],
                   preferred_element_type=jnp.float32)
    acc += jnp.dot(h_ref[0], wr_ref[0:HALF, :],
                   preferred_element_type=jnp.float32)
    acc += jnp.dot(h_ref[1], wr_ref[HALF:D, :],
                   preferred_element_type=jnp.float32)
    acc += bl_ref[0, :][None, :]
    acc = jnp.maximum(acc, 0.0)
    o_ref[0] = acc[:, 0:HALF]
    o_ref[1] = acc[:, HALF:D]


def _tc_layer(aggs, h, deg, Wl, bl, Wr):
    return pl.pallas_call(
        _tc_layer_body,
        grid=(N // BM,),
        in_specs=[
            pl.BlockSpec((2, BM, HALF), lambda i: (0, i, 0)),
            pl.BlockSpec((2, BM, HALF), lambda i: (0, i, 0)),
            pl.BlockSpec((BM, 16), lambda i: (i, 0)),
            pl.BlockSpec((D, D), lambda i: (0, 0)),
            pl.BlockSpec((1, D), lambda i: (0, 0)),
            pl.BlockSpec((D, D), lambda i: (0, 0)),
        ],
        out_specs=pl.BlockSpec((2, BM, HALF), lambda i: (0, i, 0)),
        out_shape=jax.ShapeDtypeStruct((2, N, HALF), jnp.float32),
    )(aggs, h, deg, Wl, bl, Wr)


def _tc_final_body(h_ref, w_ref, b_ref, o_ref):
    acc = jnp.dot(h_ref[0], w_ref[0:HALF, :],
                  preferred_element_type=jnp.float32)
    acc += jnp.dot(h_ref[1], w_ref[HALF:D, :],
                   preferred_element_type=jnp.float32)
    o_ref[...] = acc + b_ref[0, :][None, :]


def _tc_final(h, Wlin, blin):
    return pl.pallas_call(
        _tc_final_body,
        grid=(N // BM,),
        in_specs=[
            pl.BlockSpec((2, BM, HALF), lambda i: (0, i, 0)),
            pl.BlockSpec((D, D), lambda i: (0, 0)),
            pl.BlockSpec((1, D), lambda i: (0, 0)),
        ],
        out_specs=pl.BlockSpec((BM, D), lambda i: (i, 0)),
        out_shape=jax.ShapeDtypeStruct((N, D), jnp.float32),
    )(h, Wlin, blin)


def kernel(x, edge_index, Wl0, bl0, Wr0, Wl1, bl1, Wr1, Wl2, bl2, Wr2,
           Wlin, blin):
    src = edge_index[0]
    dst = edge_index[1]
    h = x.reshape(N, 2, HALF).transpose(1, 0, 2)
    deg = _sc_deg(dst)
    for (Wl, bl, Wr) in ((Wl0, bl0, Wr0), (Wl1, bl1, Wr1), (Wl2, bl2, Wr2)):
        aggs = _sc_agg(h.reshape(2 * N, HALF), src, dst).reshape(2, N, HALF)
        h = _tc_layer(aggs, h, deg, Wl, bl.reshape(1, D), Wr)
    return _tc_final(h, Wlin, blin.reshape(1, D))


# speedup vs baseline: 3.2759x; 3.2759x over previous
import functools

import jax
import jax.numpy as jnp
from jax import lax
from jax.experimental import pallas as pl
from jax.experimental.pallas import tpu as pltpu
from jax.experimental.pallas import tpu_sc as plsc

N = 10000
E = 160000
D = 256
HALF = 128
NSUB = 16
CH = 80
EPT = E // NSUB
NCHUNK = EPT // CH
NPAD = 10240
RPT = NPAD // NSUB
NWB = RPT // CH


def _sc_mesh():
    return plsc.VectorSubcoreMesh(core_axis_name="c", subcore_axis_name="s")


def _sc_deg(dst):
    @functools.partial(
        pl.kernel,
        out_type=jax.ShapeDtypeStruct((NPAD, HALF), jnp.float32),
        mesh=_sc_mesh(),
        scratch_types=[
            pltpu.VMEM((CH,), jnp.int32),
            pltpu.VMEM((CH, HALF), jnp.float32),
            pltpu.VMEM_SHARED((NPAD, HALF), jnp.float32),
        ],
    )
    def k(dst_hbm, out_hbm, didx_v, ones_v, acc_sh):
        c = lax.axis_index("c")
        s = lax.axis_index("s")
        t0 = s * EPT

        def zero(i, _):
            for jj in range(HALF // 16):
                ones_v[i, pl.ds(16 * jj, 16)] = jnp.zeros((16,), jnp.float32)
            return 0

        lax.fori_loop(0, CH, zero, 0)
        for t in range(NWB):
            pltpu.sync_copy(ones_v, acc_sh.at[pl.ds(s * RPT + t * CH, CH)])

        def fill(i, _):
            for jj in range(HALF // 16):
                ones_v[i, pl.ds(16 * jj, 16)] = jnp.ones((16,), jnp.float32)
            return 0

        lax.fori_loop(0, CH, fill, 0)
        plsc.subcore_barrier()

        @pl.when(c == 0)
        def _():
            def chunk(j, _):
                pltpu.sync_copy(dst_hbm.at[pl.ds(t0 + j * CH, CH)], didx_v)
                pltpu.sync_copy(ones_v, acc_sh.at[didx_v], add=True)
                return 0

            lax.fori_loop(0, NCHUNK, chunk, 0)

        plsc.subcore_barrier()

        @pl.when(c == 0)
        def _():
            for t in range(NWB):
                r0 = s * RPT + t * CH
                pltpu.sync_copy(acc_sh.at[pl.ds(r0, CH)], ones_v)
                pltpu.sync_copy(ones_v, out_hbm.at[pl.ds(r0, CH)])

    return k(dst)


def _sc_agg(h_flat, src, dst):
    @functools.partial(
        pl.kernel,
        out_type=jax.ShapeDtypeStruct((2 * NPAD, HALF), jnp.float32),
        mesh=_sc_mesh(),
        scratch_types=[
            pltpu.VMEM((CH,), jnp.int32),
            pltpu.VMEM((CH,), jnp.int32),
            pltpu.VMEM((CH, HALF), jnp.float32),
            pltpu.VMEM_SHARED((NPAD, HALF), jnp.float32),
            pltpu.SemaphoreType.DMA,
        ],
    )
    def k(h_hbm, src_hbm, dst_hbm, out_hbm, gidx_v, didx_v, rows_v,
          acc_sh, sem):
        c = lax.axis_index("c")
        s = lax.axis_index("s")
        t0 = s * EPT
        off = c * N
        oout = c * NPAD

        def zero(i, _):
            for jj in range(HALF // 16):
                rows_v[i, pl.ds(16 * jj, 16)] = jnp.zeros((16,), jnp.float32)
            return 0

        lax.fori_loop(0, CH, zero, 0)
        for t in range(NWB):
            pltpu.sync_copy(rows_v, acc_sh.at[pl.ds(s * RPT + t * CH, CH)])
        plsc.subcore_barrier()

        def chunk(j, _):
            e0 = t0 + j * CH
            pltpu.sync_copy(src_hbm.at[pl.ds(e0, CH)], gidx_v)
            pltpu.sync_copy(dst_hbm.at[pl.ds(e0, CH)], didx_v)

            def xform(kk, _):
                sl = pl.ds(16 * kk, 16)
                gidx_v[sl] = gidx_v[sl] + jnp.broadcast_to(off, (16,))
                return 0

            lax.fori_loop(0, CH // 16, xform, 0)
            pltpu.async_copy(h_hbm.at[gidx_v], rows_v, sem).wait()
            pltpu.sync_copy(rows_v, acc_sh.at[didx_v], add=True)
            return 0

        lax.fori_loop(0, NCHUNK, chunk, 0)
        plsc.subcore_barrier()

        for t in range(NWB):
            r0 = s * RPT + t * CH
            pltpu.sync_copy(acc_sh.at[pl.ds(r0, CH)], rows_v)
            pltpu.sync_copy(rows_v, out_hbm.at[pl.ds(oout + r0, CH)])

    return k(h_flat, src, dst)


BM = 1000


def _tc_layer_body(a_ref, h_ref, deg_ref, wl_ref, bl_ref, wr_ref, o_ref):
    r = 1.0 / jnp.maximum(deg_ref[:, 0:1], 1.0)
    acc = jnp.dot(a_ref[0] * r, wl_ref[0:HALF, :],
                  preferred_element_type=jnp.float32)
    acc += jnp.dot(a_ref[1] * r, wl_ref[HALF:D, :],
                   preferred_element_type=jnp.float32)
    acc += jnp.dot(h_ref[0], wr_ref[0:HALF, :],
                   preferred_element_type=jnp.float32)
    acc += jnp.dot(h_ref[1], wr_ref[HALF:D, :],
                   preferred_element_type=jnp.float32)
    acc += bl_ref[0, :][None, :]
    acc = jnp.maximum(acc, 0.0)
    o_ref[0] = acc[:, 0:HALF]
    o_ref[1] = acc[:, HALF:D]


def _tc_layer(aggs, h, deg, Wl, bl, Wr):
    return pl.pallas_call(
        _tc_layer_body,
        grid=(N // BM,),
        in_specs=[
            pl.BlockSpec((2, BM, HALF), lambda i: (0, i, 0)),
            pl.BlockSpec((2, BM, HALF), lambda i: (0, i, 0)),
            pl.BlockSpec((BM, HALF), lambda i: (i, 0)),
            pl.BlockSpec((D, D), lambda i: (0, 0)),
            pl.BlockSpec((1, D), lambda i: (0, 0)),
            pl.BlockSpec((D, D), lambda i: (0, 0)),
        ],
        out_specs=pl.BlockSpec((2, BM, HALF), lambda i: (0, i, 0)),
        out_shape=jax.ShapeDtypeStruct((2, N, HALF), jnp.float32),
    )(aggs, h, deg, Wl, bl, Wr)


def _tc_final_body(h_ref, w_ref, b_ref, o_ref):
    acc = jnp.dot(h_ref[0], w_ref[0:HALF, :],
                  preferred_element_type=jnp.float32)
    acc += jnp.dot(h_ref[1], w_ref[HALF:D, :],
                   preferred_element_type=jnp.float32)
    o_ref[...] = acc + b_ref[0, :][None, :]


def _tc_final(h, Wlin, blin):
    return pl.pallas_call(
        _tc_final_body,
        grid=(N // BM,),
        in_specs=[
            pl.BlockSpec((2, BM, HALF), lambda i: (0, i, 0)),
            pl.BlockSpec((D, D), lambda i: (0, 0)),
            pl.BlockSpec((1, D), lambda i: (0, 0)),
        ],
        out_specs=pl.BlockSpec((BM, D), lambda i: (i, 0)),
        out_shape=jax.ShapeDtypeStruct((N, D), jnp.float32),
    )(h, Wlin, blin)


def kernel(x, edge_index, Wl0, bl0, Wr0, Wl1, bl1, Wr1, Wl2, bl2, Wr2,
           Wlin, blin):
    src = edge_index[0]
    dst = edge_index[1]
    h = x.reshape(N, 2, HALF).transpose(1, 0, 2)
    deg = _sc_deg(dst)
    for (Wl, bl, Wr) in ((Wl0, bl0, Wr0), (Wl1, bl1, Wr1), (Wl2, bl2, Wr2)):
        aggs = _sc_agg(h.reshape(2 * N, HALF), src, dst).reshape(2, NPAD, HALF)
        h = _tc_layer(aggs, h, deg, Wl, bl.reshape(1, D), Wr)
    return _tc_final(h, Wlin, blin.reshape(1, D))
